# Initial kernel scaffold; baseline (speedup 1.0000x reference)
import jax
import jax.numpy as jnp
from jax.experimental import pallas as pl


def kernel(x, edge_index, batch, W1, b1, W2, b2, Wm1, bm1, Wm2, bm2):
    raise NotImplementedError("write your pallas kernel here")


# speedup vs baseline: 8.3386x; 8.3386x over previous
import functools

import jax
import jax.numpy as jnp
from jax import lax
from jax.experimental import pallas as pl
from jax.experimental.pallas import tpu as pltpu
from jax.experimental.pallas import tpu_sc as plsc

N = 10000
E = 320000
D = 128
NG = 64

NC = 2
NS = 16
NW = NC * NS

LANES = 128
EPT_ROWS = 80
E_PAD = NW * EPT_ROWS * LANES
ACC_ROWS = 10240
RPT = ACC_ROWS // NS
TRASH = N

_MESH = plsc.VectorSubcoreMesh(core_axis_name="c", subcore_axis_name="s")


_SCATTER_OUT = jax.ShapeDtypeStruct((NC, ACC_ROWS, D), jnp.float32)
_SCATTER_SCRATCH = [
    pltpu.VMEM((EPT_ROWS, LANES), jnp.int32),
    pltpu.VMEM((EPT_ROWS, LANES), jnp.int32),
    pltpu.VMEM((LANES, D), jnp.float32),
    pltpu.MemorySpace.VMEM_SHARED((ACC_ROWS, D), jnp.float32),
    pltpu.SemaphoreType.DMA,
]


def _sc_scatter_rows_body(g_hbm, src_hbm, dst_hbm, z_hbm, out_hbm,
                          src_v, dst_v, rows_v, acc, sem):
    c = lax.axis_index("c")
    s = lax.axis_index("s")
    wid = s * NC + c
    pltpu.sync_copy(z_hbm, acc.at[pl.ds(s * RPT, RPT)])
    base = wid * EPT_ROWS
    pltpu.sync_copy(src_hbm.at[pl.ds(base, EPT_ROWS)], src_v)
    pltpu.sync_copy(dst_hbm.at[pl.ds(base, EPT_ROWS)], dst_v)
    plsc.subcore_barrier()

    def body(j, carry):
        pltpu.async_copy(g_hbm.at[src_v.at[j]], rows_v, sem).wait()
        pltpu.sync_copy(rows_v, acc.at[dst_v.at[j]], add=True)
        return carry

    lax.fori_loop(0, EPT_ROWS, body, 0, unroll=False)
    plsc.subcore_barrier()
    pltpu.sync_copy(acc.at[pl.ds(s * RPT, RPT)],
                    out_hbm.at[c].at[pl.ds(s * RPT, RPT)])


_sc_scatter_rows = functools.partial(
    pl.kernel, out_type=_SCATTER_OUT, mesh=_MESH,
    scratch_types=_SCATTER_SCRATCH)(_sc_scatter_rows_body)


_DEG_OUT = jax.ShapeDtypeStruct((NC, ACC_ROWS, D), jnp.float32)
_DEG_SCRATCH = [
    pltpu.VMEM((EPT_ROWS, LANES), jnp.int32),
    pltpu.VMEM((LANES, D), jnp.float32),
    pltpu.MemorySpace.VMEM_SHARED((ACC_ROWS, D), jnp.float32),
]


def _sc_degree_body(dst_hbm, ones_hbm, z_hbm, out_hbm, dst_v, ones_v, acc):
    c = lax.axis_index("c")
    s = lax.axis_index("s")
    wid = s * NC + c
    pltpu.sync_copy(z_hbm, acc.at[pl.ds(s * RPT, RPT)])
    pltpu.sync_copy(ones_hbm, ones_v)
    pltpu.sync_copy(dst_hbm.at[pl.ds(wid * EPT_ROWS, EPT_ROWS)], dst_v)
    plsc.subcore_barrier()

    def body(j, carry):
        pltpu.sync_copy(ones_v, acc.at[dst_v.at[j]], add=True)
        return carry

    lax.fori_loop(0, EPT_ROWS, body, 0, unroll=False)
    plsc.subcore_barrier()
    pltpu.sync_copy(acc.at[pl.ds(s * RPT, RPT)],
                    out_hbm.at[c].at[pl.ds(s * RPT, RPT)])


_sc_degree = functools.partial(
    pl.kernel, out_type=_DEG_OUT, mesh=_MESH,
    scratch_types=_DEG_SCRATCH)(_sc_degree_body)


_BLK = 1000


def _dinv_block(dega_ref, degb_ref):
    deg = dega_ref[:, 0:1] + degb_ref[:, 0:1] + 1.0
    return lax.rsqrt(deg)


def _tc_g1_body(dega_ref, degb_ref, x_ref, w_ref, g_ref):
    dinv = _dinv_block(dega_ref, degb_ref)
    g_ref[...] = dinv * jnp.dot(x_ref[...], w_ref[...],
                                preferred_element_type=jnp.float32)


def _tc_g1(dega, degb, x, W1):
    return pl.pallas_call(
        _tc_g1_body,
        grid=(N // _BLK,),
        in_specs=[
            pl.BlockSpec((_BLK, 16), lambda i: (i, 0)),
            pl.BlockSpec((_BLK, 16), lambda i: (i, 0)),
            pl.BlockSpec((_BLK, D), lambda i: (i, 0)),
            pl.BlockSpec((D, D), lambda i: (0, 0)),
        ],
        out_specs=pl.BlockSpec((_BLK, D), lambda i: (i, 0)),
        out_shape=jax.ShapeDtypeStruct((N, D), jnp.float32),
    )(dega, degb, x, W1)


def _tc_g2_body(dega_ref, degb_ref, s0_ref, s1_ref, g1_ref, b1_ref, w2_ref,
                g2_ref):
    dinv = _dinv_block(dega_ref, degb_ref)
    h1 = dinv * (s0_ref[...] + s1_ref[...] + g1_ref[...]) + b1_ref[...]
    h1 = jnp.maximum(h1, 0.0)
    g2_ref[...] = dinv * jnp.dot(h1, w2_ref[...],
                                 preferred_element_type=jnp.float32)


def _tc_g2(dega, degb, s0, s1, g1, b1r, W2):
    return pl.pallas_call(
        _tc_g2_body,
        grid=(N // _BLK,),
        in_specs=[
            pl.BlockSpec((_BLK, 16), lambda i: (i, 0)),
            pl.BlockSpec((_BLK, 16), lambda i: (i, 0)),
            pl.BlockSpec((_BLK, D), lambda i: (i, 0)),
            pl.BlockSpec((_BLK, D), lambda i: (i, 0)),
            pl.BlockSpec((_BLK, D), lambda i: (i, 0)),
            pl.BlockSpec((1, D), lambda i: (0, 0)),
            pl.BlockSpec((D, D), lambda i: (0, 0)),
        ],
        out_specs=pl.BlockSpec((_BLK, D), lambda i: (i, 0)),
        out_shape=jax.ShapeDtypeStruct((N, D), jnp.float32),
    )(dega, degb, s0, s1, g1, b1r, W2)


def _tc_head_body(dega_ref, degb_ref, s0_ref, s1_ref, g2_ref, b2_ref,
                  batch_ref, wm1_ref, bm1_ref, wm2_ref, bm2_ref, out_ref):
    deg = dega_ref[:, 0:1] + degb_ref[:, 0:1] + 1.0
    dinv = lax.rsqrt(deg)
    h2 = dinv * (s0_ref[...] + s1_ref[...] + g2_ref[...]) + b2_ref[...]
    gid = lax.broadcasted_iota(jnp.int32, (NG, N), 0).astype(jnp.float32)
    onehot = (batch_ref[...] == gid).astype(jnp.float32)
    sums = jnp.dot(onehot, h2, preferred_element_type=jnp.float32)
    counts = jnp.sum(onehot, axis=1, keepdims=True)
    pooled = sums / jnp.maximum(counts, 1.0)
    z = jnp.dot(pooled, wm1_ref[...], preferred_element_type=jnp.float32)
    z = jnp.maximum(z + bm1_ref[...], 0.0)
    out_ref[...] = (jnp.sum(z * wm2_ref[...], axis=1, keepdims=True)
                    + bm2_ref[...])


def _tc_head(dega, degb, s0, s1, g2, b2r, batchf, Wm1, bm1r, wm2r, bm2r):
    return pl.pallas_call(
        _tc_head_body,
        out_shape=jax.ShapeDtypeStruct((NG, 1), jnp.float32),
    )(dega, degb, s0, s1, g2, b2r, batchf, Wm1, bm1r, wm2r, bm2r)


def kernel(x, edge_index, batch, W1, b1, W2, b2, Wm1, bm1, Wm2, bm2):
    src = edge_index[0]
    dst = edge_index[1]
    pad = E_PAD - E
    src_p = jnp.concatenate(
        [src, jnp.zeros((pad,), jnp.int32)]).reshape(-1, LANES)
    dst_p = jnp.concatenate(
        [dst, jnp.full((pad,), TRASH, jnp.int32)]).reshape(-1, LANES)
    zeros_d = jnp.zeros((RPT, D), jnp.float32)
    ones_d = jnp.ones((LANES, D), jnp.float32)

    degp = _sc_degree(dst_p, ones_d, zeros_d)
    dega = degp[0, :N, 0:16]
    degb = degp[1, :N, 0:16]

    g1 = _tc_g1(dega, degb, x, W1)
    S1 = _sc_scatter_rows(g1, src_p, dst_p, zeros_d)
    g2 = _tc_g2(dega, degb, S1[0, :N], S1[1, :N], g1,
                b1.reshape(1, D), W2)
    S2 = _sc_scatter_rows(g2, src_p, dst_p, zeros_d)
    out = _tc_head(dega, degb, S2[0, :N], S2[1, :N], g2,
                   b2.reshape(1, D),
                   batch.astype(jnp.float32).reshape(1, N),
                   Wm1, bm1.reshape(1, 16),
                   Wm2.reshape(1, 16), bm2.reshape(1, 1))
    return out.reshape(-1)
